# baseline (speedup 1.0000x reference)
import jax
import jax.numpy as jnp
from jax import lax
from jax.experimental import pallas as pl
from jax.experimental.pallas import tpu as pltpu
from jax.experimental.pallas import tpu_sc as plsc

B, L = 1024, 512
VOCAB, D = 257, 128
VP = 384
NC, NS = 2, 16
NW = NC * NS
RW = B // NW


def _sc_hist_body(x_hbm, zeros_hbm, out_hbm, x_v, counts_v):
    wid = lax.axis_index("s") * NC + lax.axis_index("c")
    base = wid * RW
    pltpu.sync_copy(x_hbm.at[pl.ds(base, RW)], x_v)
    pltpu.sync_copy(zeros_hbm, counts_v)
    lanes = lax.iota(jnp.int32, 16)
    inv_l = jnp.full((16,), 1.0 / L, jnp.float32)
    for g in range(RW // 16):
        rows = lanes + (g * 16)
        row_c = rows * VP

        @plsc.parallel_loop(0, L - 16, unroll=8, carry=lanes)
        def _(l, col, rows=rows, row_c=row_c):
            idx = plsc.load_gather(x_v, [rows, col])
            plsc.addupdate_scatter(counts_v, [row_c + idx], inv_l)
            return col + 1

        @plsc.parallel_loop(L - 16, L, carry=None)
        def _(l, rows=rows, row_c=row_c):
            col = lanes + l
            col = jnp.where(col >= L, col - L, col)
            idx = plsc.load_gather(x_v, [rows, col])
            plsc.addupdate_scatter(counts_v, [row_c + idx], inv_l)
    pltpu.sync_copy(counts_v, out_hbm.at[pl.ds(base * VP, RW * VP)])


_sc_hist = pl.kernel(
    _sc_hist_body,
    out_type=jax.ShapeDtypeStruct((B * VP,), jnp.float32),
    mesh=plsc.VectorSubcoreMesh(core_axis_name="c", subcore_axis_name="s"),
    scratch_types=[
        pltpu.VMEM((RW, L), jnp.int32),
        pltpu.VMEM((RW * VP,), jnp.float32),
    ],
    compiler_params=pltpu.CompilerParams(needs_layout_passes=False),
)


def _dot_f32(a, b):
    bf16, f32 = jnp.bfloat16, jnp.float32
    a_hi = a.astype(bf16)
    a_lo = (a - a_hi.astype(f32)).astype(bf16)
    b_hi = b.astype(bf16)
    b_lo = (b - b_hi.astype(f32)).astype(bf16)
    dims = (((1,), (0,)), ((), ()))

    def mm(p, q):
        return jax.lax.dot_general(p, q, dims, preferred_element_type=f32)

    return ((mm(a_lo, b_lo) + mm(a_lo, b_hi)) +
            (mm(a_hi, b_lo) + mm(a_hi, b_hi)))


def _dot_bf16(a, b):
    return jax.lax.dot_general(a.astype(jnp.bfloat16), b.astype(jnp.bfloat16),
                               (((1,), (0,)), ((), ())),
                               preferred_element_type=jnp.float32)


def _mlp_body(counts_ref, table_ref, w1_ref, b1_ref, w2_ref, b2_ref,
              w3_ref, b3_ref, w4_ref, b4_ref, out_ref):
    pooled = _dot_f32(counts_ref[...], table_ref[...])
    h = jnp.maximum(_dot_bf16(pooled, w1_ref[...]) + b1_ref[...], 0.0)
    h = jnp.maximum(_dot_bf16(h, w2_ref[...]) + b2_ref[...], 0.0)
    h = jnp.maximum(_dot_bf16(h, w3_ref[...]) + b3_ref[...], 0.0)
    out_ref[...] = _dot_bf16(h, w4_ref[...]) + b4_ref[...]


_mlp = pl.pallas_call(
    _mlp_body,
    out_shape=jax.ShapeDtypeStruct((B, 2), jnp.float32),
)


def kernel(x, table, W1, b1, W2, b2, W3, b3, W4, b4):
    x = x.astype(jnp.int32)
    zeros = jnp.zeros((RW * VP,), jnp.float32)
    counts = _sc_hist(x, zeros).reshape(B, VP)
    table_p = jnp.pad(table, ((0, VP - VOCAB), (0, 0)))
    return _mlp(counts, table_p, W1, b1[None, :], W2, b2[None, :],
                W3, b3[None, :], W4, b4[None, :])


# speedup vs baseline: 1.0037x; 1.0037x over previous
import jax
import jax.numpy as jnp
from jax import lax
from jax.experimental import pallas as pl
from jax.experimental.pallas import tpu as pltpu
from jax.experimental.pallas import tpu_sc as plsc

B, L = 1024, 512
VOCAB, D = 257, 128
VP = 264
NC, NS = 2, 16
NW = NC * NS
RW = B // NW


def _sc_hist_body(x_hbm, zeros_hbm, out_hbm, x_v, counts_v, sem_x, sem_z):
    wid = lax.axis_index("s") * NC + lax.axis_index("c")
    base = wid * RW
    cp_x = pltpu.async_copy(x_hbm.at[pl.ds(base, RW)], x_v, sem_x)
    cp_z = pltpu.async_copy(zeros_hbm, counts_v, sem_z)
    cp_x.wait()
    cp_z.wait()
    lanes = lax.iota(jnp.int32, 16)
    inv_l = jnp.full((16,), 1.0 / L, jnp.float32)
    for g in range(RW // 16):
        rows = lanes + (g * 16)
        row_c = rows * VP

        @plsc.parallel_loop(0, L - 16, unroll=8, carry=lanes)
        def _(l, col, rows=rows, row_c=row_c):
            idx = plsc.load_gather(x_v, [rows, col])
            plsc.addupdate_scatter(counts_v, [row_c + idx], inv_l)
            return col + 1

        @plsc.parallel_loop(L - 16, L, carry=None)
        def _(l, rows=rows, row_c=row_c):
            col = lanes + l
            col = jnp.where(col >= L, col - L, col)
            idx = plsc.load_gather(x_v, [rows, col])
            plsc.addupdate_scatter(counts_v, [row_c + idx], inv_l)
    pltpu.sync_copy(counts_v, out_hbm.at[pl.ds(base * VP, RW * VP)])


_sc_hist = pl.kernel(
    _sc_hist_body,
    out_type=jax.ShapeDtypeStruct((B * VP,), jnp.float32),
    mesh=plsc.VectorSubcoreMesh(core_axis_name="c", subcore_axis_name="s"),
    scratch_types=[
        pltpu.VMEM((RW, L), jnp.int32),
        pltpu.VMEM((RW * VP,), jnp.float32),
        pltpu.SemaphoreType.DMA,
        pltpu.SemaphoreType.DMA,
    ],
    compiler_params=pltpu.CompilerParams(needs_layout_passes=False),
)


def _dot_f32(a, b):
    bf16, f32 = jnp.bfloat16, jnp.float32
    a_hi = a.astype(bf16)
    a_lo = (a - a_hi.astype(f32)).astype(bf16)
    b_hi = b.astype(bf16)
    b_lo = (b - b_hi.astype(f32)).astype(bf16)
    dims = (((1,), (0,)), ((), ()))

    def mm(p, q):
        return jax.lax.dot_general(p, q, dims, preferred_element_type=f32)

    return ((mm(a_lo, b_lo) + mm(a_lo, b_hi)) +
            (mm(a_hi, b_lo) + mm(a_hi, b_hi)))


def _dot_bf16(a, b):
    return jax.lax.dot_general(a.astype(jnp.bfloat16), b.astype(jnp.bfloat16),
                               (((1,), (0,)), ((), ())),
                               preferred_element_type=jnp.float32)


def _mlp_body(counts_ref, table_ref, w1_ref, b1_ref, w2_ref, b2_ref,
              w3_ref, b3_ref, w4_ref, b4_ref, out_ref):
    pooled = _dot_f32(counts_ref[...], table_ref[...])
    h = jnp.maximum(_dot_bf16(pooled, w1_ref[...]) + b1_ref[...], 0.0)
    h = jnp.maximum(_dot_bf16(h, w2_ref[...]) + b2_ref[...], 0.0)
    h = jnp.maximum(_dot_bf16(h, w3_ref[...]) + b3_ref[...], 0.0)
    out_ref[...] = _dot_bf16(h, w4_ref[...]) + b4_ref[...]


_mlp = pl.pallas_call(
    _mlp_body,
    out_shape=jax.ShapeDtypeStruct((B, 2), jnp.float32),
)


def kernel(x, table, W1, b1, W2, b2, W3, b3, W4, b4):
    x = x.astype(jnp.int32)
    zeros = jnp.zeros((RW * VP,), jnp.float32)
    counts = _sc_hist(x, zeros).reshape(B, VP)
    table_p = jnp.pad(table, ((0, VP - VOCAB), (0, 0)))
    return _mlp(counts, table_p, W1, b1[None, :], W2, b2[None, :],
                W3, b3[None, :], W4, b4[None, :])


# speedup vs baseline: 1.0060x; 1.0024x over previous
import jax
import jax.numpy as jnp
from jax import lax
from jax.experimental import pallas as pl
from jax.experimental.pallas import tpu as pltpu
from jax.experimental.pallas import tpu_sc as plsc

B, L = 1024, 512
VOCAB, D = 257, 128
VP = 264
NC, NS = 2, 16
NW = NC * NS
RW = B // NW


def _sc_hist_body(x_hbm, zeros_hbm, out_hbm, x_v, counts_v, sem_x, sem_z):
    wid = lax.axis_index("s") * NC + lax.axis_index("c")
    base = wid * RW
    cp_x = pltpu.async_copy(x_hbm.at[pl.ds(base, RW)], x_v, sem_x)
    cp_z = pltpu.async_copy(zeros_hbm, counts_v, sem_z)
    cp_x.wait()
    cp_z.wait()
    lanes = lax.iota(jnp.int32, 16)
    inv_l = jnp.full((16,), 1.0 / L, jnp.float32)
    grp = [(lanes + g * 16, (lanes + g * 16) * VP) for g in range(RW // 16)]

    @plsc.parallel_loop(0, L - 16, unroll=4, carry=lanes)
    def _(l, col):
        for rows, row_c in grp:
            idx = plsc.load_gather(x_v, [rows, col])
            plsc.addupdate_scatter(counts_v, [row_c + idx], inv_l)
        return col + 1

    @plsc.parallel_loop(L - 16, L, carry=None)
    def _(l):
        col = lanes + l
        col = jnp.where(col >= L, col - L, col)
        for rows, row_c in grp:
            idx = plsc.load_gather(x_v, [rows, col])
            plsc.addupdate_scatter(counts_v, [row_c + idx], inv_l)
    pltpu.sync_copy(counts_v, out_hbm.at[pl.ds(base * VP, RW * VP)])


_sc_hist = pl.kernel(
    _sc_hist_body,
    out_type=jax.ShapeDtypeStruct((B * VP,), jnp.float32),
    mesh=plsc.VectorSubcoreMesh(core_axis_name="c", subcore_axis_name="s"),
    scratch_types=[
        pltpu.VMEM((RW, L), jnp.int32),
        pltpu.VMEM((RW * VP,), jnp.float32),
        pltpu.SemaphoreType.DMA,
        pltpu.SemaphoreType.DMA,
    ],
    compiler_params=pltpu.CompilerParams(needs_layout_passes=False),
)


def _dot_f32(a, b):
    bf16, f32 = jnp.bfloat16, jnp.float32
    a_hi = a.astype(bf16)
    a_lo = (a - a_hi.astype(f32)).astype(bf16)
    b_hi = b.astype(bf16)
    b_lo = (b - b_hi.astype(f32)).astype(bf16)
    dims = (((1,), (0,)), ((), ()))

    def mm(p, q):
        return jax.lax.dot_general(p, q, dims, preferred_element_type=f32)

    return ((mm(a_lo, b_lo) + mm(a_lo, b_hi)) +
            (mm(a_hi, b_lo) + mm(a_hi, b_hi)))


def _dot_bf16(a, b):
    return jax.lax.dot_general(a.astype(jnp.bfloat16), b.astype(jnp.bfloat16),
                               (((1,), (0,)), ((), ())),
                               preferred_element_type=jnp.float32)


def _mlp_body(counts_ref, table_ref, w1_ref, b1_ref, w2_ref, b2_ref,
              w3_ref, b3_ref, w4_ref, b4_ref, out_ref):
    pooled = _dot_f32(counts_ref[...], table_ref[...])
    h = jnp.maximum(_dot_bf16(pooled, w1_ref[...]) + b1_ref[...], 0.0)
    h = jnp.maximum(_dot_bf16(h, w2_ref[...]) + b2_ref[...], 0.0)
    h = jnp.maximum(_dot_bf16(h, w3_ref[...]) + b3_ref[...], 0.0)
    out_ref[...] = _dot_bf16(h, w4_ref[...]) + b4_ref[...]


_mlp = pl.pallas_call(
    _mlp_body,
    out_shape=jax.ShapeDtypeStruct((B, 2), jnp.float32),
)


def kernel(x, table, W1, b1, W2, b2, W3, b3, W4, b4):
    x = x.astype(jnp.int32)
    zeros = jnp.zeros((RW * VP,), jnp.float32)
    counts = _sc_hist(x, zeros).reshape(B, VP)
    table_p = jnp.pad(table, ((0, VP - VOCAB), (0, 0)))
    return _mlp(counts, table_p, W1, b1[None, :], W2, b2[None, :],
                W3, b3[None, :], W4, b4[None, :])
